# baseline (speedup 1.0000x reference)
import functools

import jax
import jax.numpy as jnp
from jax import lax
from jax.experimental import pallas as pl
from jax.experimental.pallas import tpu as pltpu
from jax.experimental.pallas import tpu_sc as plsc

_LANES = 16


@functools.lru_cache(maxsize=None)
def _make_jitter_kernel(n_rows, n_in, n_out, rows_per_batch):
    NC = 2
    NS = 16
    NW = NC * NS
    assert n_rows % NW == 0
    rows_per_tile = n_rows // NW
    RB = 8
    assert rows_per_tile % RB == 0
    assert rows_per_batch % rows_per_tile == 0
    n_blocks = rows_per_tile // RB
    n_vec = n_out // _LANES
    assert n_out % _LANES == 0

    mesh = plsc.VectorSubcoreMesh(core_axis_name="c", subcore_axis_name="s")

    @functools.partial(
        pl.kernel,
        out_type=jax.ShapeDtypeStruct((n_rows, n_out), jnp.float32),
        mesh=mesh,
        compiler_params=pltpu.CompilerParams(needs_layout_passes=False),
        scratch_types=[
            pltpu.VMEM((n_in,), jnp.int32),
            pltpu.VMEM((RB, n_in), jnp.float32),
            pltpu.VMEM((RB, n_in), jnp.float32),
            pltpu.VMEM((RB, n_out), jnp.float32),
            pltpu.VMEM((RB, n_out), jnp.float32),
            pltpu.SemaphoreType.DMA,
            pltpu.SemaphoreType.DMA,
            pltpu.SemaphoreType.DMA,
            pltpu.SemaphoreType.DMA,
        ],
    )
    def jitter(x_hbm, idx_hbm, out_hbm, idx_v, xbuf0, xbuf1, obuf0, obuf1,
               sem_i0, sem_i1, sem_o0, sem_o1):
        wid = lax.axis_index("s") * NC + lax.axis_index("c")
        row0 = wid * rows_per_tile
        batch = row0 // rows_per_batch
        pltpu.sync_copy(idx_hbm.at[batch], idx_v)

        xbufs, obufs = (xbuf0, xbuf1), (obuf0, obuf1)
        sems_i, sems_o = (sem_i0, sem_i1), (sem_o0, sem_o1)

        def start_in(blk):
            row = row0 + blk * RB
            return pltpu.async_copy(
                x_hbm.at[pl.ds(row, RB)], xbufs[blk % 2], sems_i[blk % 2])

        def start_out(blk):
            row = row0 + blk * RB
            return pltpu.async_copy(
                obufs[blk % 2], out_hbm.at[pl.ds(row, RB)], sems_o[blk % 2])

        in_h = {0: start_in(0)}
        out_h = {}
        for blk in range(n_blocks):
            if blk + 1 < n_blocks:
                in_h[blk + 1] = start_in(blk + 1)
            in_h.pop(blk).wait()
            if blk >= 2:
                out_h.pop(blk - 2).wait()
            xbuf, obuf = xbufs[blk % 2], obufs[blk % 2]

            @plsc.parallel_loop(0, n_vec, unroll=8)
            def gather_body(ti):
                t = ti * _LANES
                iv = idx_v[pl.ds(t + 1, _LANES)]
                for r in range(RB):
                    rsplat = jnp.full((_LANES,), r, jnp.int32)
                    obuf[r, pl.ds(t, _LANES)] = plsc.load_gather(
                        xbuf, [rsplat, iv])

            out_h[blk] = start_out(blk)
        for blk in sorted(out_h):
            out_h.pop(blk).wait()

    return jitter


def kernel(x, mindex):
    B, C, T2 = x.shape
    T = T2 - 2
    idx = mindex if mindex.dtype == jnp.int32 else mindex.astype(jnp.int32)
    x2 = x.reshape(B * C, T2)
    out = _make_jitter_kernel(B * C, T2, T, C)(x2, idx)
    return out.reshape(B, C, T)


# speedup vs baseline: 1.0122x; 1.0122x over previous
import functools

import jax
import jax.numpy as jnp
from jax import lax
from jax.experimental import pallas as pl
from jax.experimental.pallas import tpu as pltpu
from jax.experimental.pallas import tpu_sc as plsc

_LANES = 16


@functools.lru_cache(maxsize=None)
def _make_jitter_kernel(n_rows, n_in, n_out, rows_per_batch):
    NC = 2
    NS = 16
    NW = NC * NS
    assert n_rows % NW == 0
    rows_per_tile = n_rows // NW
    RB = 8
    assert rows_per_tile % RB == 0
    assert rows_per_batch % rows_per_tile == 0
    n_blocks = rows_per_tile // RB
    n_vec = n_out // _LANES
    assert n_out % _LANES == 0

    mesh = plsc.VectorSubcoreMesh(core_axis_name="c", subcore_axis_name="s")

    @functools.partial(
        pl.kernel,
        out_type=jax.ShapeDtypeStruct((n_rows, n_out), jnp.float32),
        mesh=mesh,
        compiler_params=pltpu.CompilerParams(needs_layout_passes=False),
        scratch_types=[
            pltpu.VMEM((n_in,), jnp.int32),
            pltpu.VMEM((RB, n_in), jnp.float32),
            pltpu.VMEM((RB, n_in), jnp.float32),
            pltpu.VMEM((RB, n_out), jnp.float32),
            pltpu.VMEM((RB, n_out), jnp.float32),
            pltpu.SemaphoreType.DMA,
            pltpu.SemaphoreType.DMA,
            pltpu.SemaphoreType.DMA,
            pltpu.SemaphoreType.DMA,
        ],
    )
    def jitter(x_hbm, idx_hbm, out_hbm, idx_v, xbuf0, xbuf1, obuf0, obuf1,
               sem_i0, sem_i1, sem_o0, sem_o1):
        wid = lax.axis_index("s") * NC + lax.axis_index("c")
        row0 = wid * rows_per_tile
        batch = row0 // rows_per_batch
        pltpu.sync_copy(idx_hbm.at[batch], idx_v)

        xbufs, obufs = (xbuf0, xbuf1), (obuf0, obuf1)
        sems_i, sems_o = (sem_i0, sem_i1), (sem_o0, sem_o1)

        def start_in(blk):
            row = row0 + blk * RB
            return pltpu.async_copy(
                x_hbm.at[pl.ds(row, RB)], xbufs[blk % 2], sems_i[blk % 2])

        def start_out(blk):
            row = row0 + blk * RB
            return pltpu.async_copy(
                obufs[blk % 2], out_hbm.at[pl.ds(row, RB)], sems_o[blk % 2])

        in_h = {0: start_in(0)}
        out_h = {}
        for blk in range(n_blocks):
            if blk + 1 < n_blocks:
                in_h[blk + 1] = start_in(blk + 1)
            in_h.pop(blk).wait()
            if blk >= 2:
                out_h.pop(blk - 2).wait()
            xbuf, obuf = xbufs[blk % 2], obufs[blk % 2]

            @plsc.parallel_loop(0, n_vec, unroll=4)
            def gather_body(ti):
                t = ti * _LANES
                iv = idx_v[pl.ds(t + 1, _LANES)]
                for r in range(RB):
                    rsplat = jnp.full((_LANES,), r, jnp.int32)
                    obuf[r, pl.ds(t, _LANES)] = plsc.load_gather(
                        xbuf, [rsplat, iv])

            out_h[blk] = start_out(blk)
        for blk in sorted(out_h):
            out_h.pop(blk).wait()

    return jitter


def kernel(x, mindex):
    B, C, T2 = x.shape
    T = T2 - 2
    idx = mindex if mindex.dtype == jnp.int32 else mindex.astype(jnp.int32)
    x2 = x.reshape(B * C, T2)
    out = _make_jitter_kernel(B * C, T2, T, C)(x2, idx)
    return out.reshape(B, C, T)
